# baseline (speedup 1.0000x reference)
import functools

import jax
import jax.numpy as jnp
from jax import lax
from jax.experimental import pallas as pl
from jax.experimental.pallas import tpu as pltpu
from jax.experimental.pallas import tpu_sc as plsc

N = 1024
L = 16
NCLOUD = 16
BIG = 1e30
EPS = 1e-12


def _sc_body(px_hbm, py_hbm, pz_hbm, out_hbm,
             px_v, py_v, pz_v, sq_v, sqv_v, mind_v, da_v, db_v, lane_v):
    wid = lax.axis_index("s") * 2 + lax.axis_index("c")

    @pl.when(wid < NCLOUD)
    def _():
        iota = lax.iota(jnp.int32, L)
        lane0 = iota == 0
        bigv = jnp.full((L,), BIG, jnp.float32)

        row = pl.ds(wid * N, N)
        pltpu.sync_copy(px_hbm.at[row], px_v)
        pltpu.sync_copy(py_hbm.at[row], py_v)
        pltpu.sync_copy(pz_hbm.at[row], pz_v)

        def _bf16_rne(v):
            b = plsc.bitcast(v, jnp.uint32)
            r = (b + 0x7FFF + ((b >> 16) & 1)) & jnp.uint32(0xFFFF0000)
            return plsc.bitcast(r, jnp.float32)

        @plsc.parallel_loop(0, N, L, unroll=4)
        def _prep(i):
            x = px_v[pl.ds(i, L)]
            y = py_v[pl.ds(i, L)]
            z = pz_v[pl.ds(i, L)]
            s = x * x + y * y + z * z
            sq_v[pl.ds(i, L)] = s
            sqv_v[pl.ds(i, L)] = s
            mind_v[pl.ds(i, L)] = bigv
            xb = plsc.bitcast(_bf16_rne(x), jnp.uint32)
            yb = plsc.bitcast(_bf16_rne(y), jnp.uint32)
            px_v[pl.ds(i, L)] = plsc.bitcast(
                (xb >> 16) | (yb & jnp.uint32(0xFFFF0000)), jnp.float32)
            pz_v[pl.ds(i, L)] = _bf16_rne(z)

        def step(t, jv):
            plsc.store_scatter(mind_v, [jv], bigv, mask=lane0)
            plsc.store_scatter(sqv_v, [jv], bigv, mask=lane0)
            pj = plsc.bitcast(plsc.load_gather(px_v, [jv]), jnp.uint32)
            xj = plsc.bitcast(pj << 16, jnp.float32)
            yj = plsc.bitcast(pj & jnp.uint32(0xFFFF0000), jnp.float32)
            zj = plsc.load_gather(pz_v, [jv])
            sj = plsc.load_gather(sq_v, [jv])
            x2 = xj + xj
            y2 = yj + yj
            z2 = zj + zj

            @plsc.parallel_loop(
                0, N, L, unroll=4,
                carry=(bigv, jnp.zeros((L,), jnp.int32)))
            def sweep(i, carry):
                bv, bi = carry
                pxy = plsc.bitcast(px_v[pl.ds(i, L)], jnp.uint32)
                x = plsc.bitcast(pxy << 16, jnp.float32)
                y = plsc.bitcast(pxy & jnp.uint32(0xFFFF0000), jnp.float32)
                z = pz_v[pl.ds(i, L)]
                sv = sqv_v[pl.ds(i, L)]
                m = mind_v[pl.ds(i, L)]
                d2 = (sv + sj) - (x * x2 + y * y2 + z * z2)
                m2 = jnp.minimum(m, d2)
                mind_v[pl.ds(i, L)] = m2
                better = m2 < bv
                bv = jnp.minimum(bv, m2)
                bi = jnp.where(better, jnp.full((L,), i, jnp.int32), bi)
                return bv, bi

            bv, bi = sweep
            m = jnp.min(bv)
            lane_v[...] = bi
            lane = plsc.all_reduce_ffs(bv == m)
            if lane.shape != (L,):
                lane = jnp.broadcast_to(lane, (L,))
            j = plsc.load_gather(lane_v, [lane]) + lane
            death = jnp.maximum(m, jnp.float32(EPS))
            plsc.store_scatter(
                da_v, [jnp.full((L,), t, jnp.int32)],
                jnp.broadcast_to(death, (L,)), mask=lane0)
            return j

        lax.fori_loop(0, N - 1, step, jnp.zeros((L,), jnp.int32))
        plsc.store_scatter(
            da_v, [jnp.full((L,), N - 1, jnp.int32)], bigv,
            mask=lane0)

        def make_stage(src, dst, k, s):
            def body(i):
                g = jnp.full((L,), i, jnp.int32) + iota
                p = lax.bitwise_xor(g, jnp.int32(s))
                ag = src[pl.ds(i, L)]
                ap = plsc.load_gather(src, [p])
                up = lax.bitwise_and(g, jnp.int32(k)) == 0
                lowr = lax.bitwise_and(g, jnp.int32(s)) == 0
                tmin = up == lowr
                dst[pl.ds(i, L)] = jnp.where(
                    tmin, jnp.minimum(ag, ap), jnp.maximum(ag, ap))
            return body

        src, dst = da_v, db_v
        for kk in range(1, 11):
            k = 1 << kk
            for ss in range(kk - 1, -1, -1):
                plsc.parallel_loop(0, N, L, unroll=2)(
                    make_stage(src, dst, k, 1 << ss))
                src, dst = dst, src

        pltpu.sync_copy(src, out_hbm.at[row])


_sc_prim_sort = functools.partial(
    pl.kernel,
    mesh=plsc.VectorSubcoreMesh(core_axis_name="c", subcore_axis_name="s"),
    out_type=jax.ShapeDtypeStruct((NCLOUD * N,), jnp.float32),
    compiler_params=pltpu.CompilerParams(needs_layout_passes=False),
    scratch_types=[
        pltpu.VMEM((N,), jnp.float32),
        pltpu.VMEM((N,), jnp.float32),
        pltpu.VMEM((N,), jnp.float32),
        pltpu.VMEM((N,), jnp.float32),
        pltpu.VMEM((N,), jnp.float32),
        pltpu.VMEM((N,), jnp.float32),
        pltpu.VMEM((N,), jnp.float32),
        pltpu.VMEM((N,), jnp.float32),
        pltpu.VMEM((L,), jnp.int32),
    ],
)(_sc_body)


def _finish_body(d2_ref, out_ref):
    d = jnp.sqrt(d2_ref[...])
    diff = d[0:8, :] - d[8:16, :]
    tl = jnp.sqrt(jnp.sum(diff * diff, axis=1))
    out_ref[...] = jnp.mean(tl).reshape(1, 1)


def kernel(gts, preds):
    pts = jnp.concatenate([gts, preds], 0)
    px = pts[:, :, 0].reshape(-1)
    py = pts[:, :, 1].reshape(-1)
    pz = pts[:, :, 2].reshape(-1)
    sorted_d2 = _sc_prim_sort(px, py, pz).reshape(NCLOUD, N)
    out = pl.pallas_call(
        _finish_body,
        out_shape=jax.ShapeDtypeStruct((1, 1), jnp.float32),
    )(sorted_d2)
    return out[0, 0]


# speedup vs baseline: 1.0365x; 1.0365x over previous
import functools

import jax
import jax.numpy as jnp
from jax import lax
from jax.experimental import pallas as pl
from jax.experimental.pallas import tpu as pltpu
from jax.experimental.pallas import tpu_sc as plsc

N = 1024
L = 16
NCLOUD = 16
BIG = 1e30
EPS = 1e-12


def _sc_body(px_hbm, py_hbm, pz_hbm, out_hbm,
             px_v, py_v, pz_v, sq_v, fr_v, da_v, db_v, lane_v):
    wid = lax.axis_index("s") * 2 + lax.axis_index("c")

    @pl.when(wid < NCLOUD)
    def _():
        iota = lax.iota(jnp.int32, L)
        lane0 = iota == 0
        bigv = jnp.full((L,), BIG, jnp.float32)

        row = pl.ds(wid * N, N)
        pltpu.sync_copy(px_hbm.at[row], px_v)
        pltpu.sync_copy(py_hbm.at[row], py_v)
        pltpu.sync_copy(pz_hbm.at[row], pz_v)

        def _bf16_rne(v):
            b = plsc.bitcast(v, jnp.uint32)
            r = (b + 0x7FFF + ((b >> 16) & 1)) & jnp.uint32(0xFFFF0000)
            return plsc.bitcast(r, jnp.float32)

        @plsc.parallel_loop(0, N, L, unroll=4)
        def _prep(i):
            x = px_v[pl.ds(i, L)]
            y = py_v[pl.ds(i, L)]
            z = pz_v[pl.ds(i, L)]
            s = x * x + y * y + z * z
            sq_v[pl.ds(i, L)] = s
            fr_v[pl.ds(i, L)] = s
            fr_v[pl.ds(N + i, L)] = bigv
            xb = plsc.bitcast(_bf16_rne(x), jnp.uint32)
            yb = plsc.bitcast(_bf16_rne(y), jnp.uint32)
            px_v[pl.ds(i, L)] = plsc.bitcast(
                (xb >> 16) | (yb & jnp.uint32(0xFFFF0000)), jnp.float32)
            pz_v[pl.ds(i, L)] = _bf16_rne(z)

        def step(t, jv):
            plsc.store_scatter(
                fr_v, [jnp.where(lane0, jv + N, jv)], bigv, mask=iota < 2)
            pj = plsc.bitcast(plsc.load_gather(px_v, [jv]), jnp.uint32)
            xj = plsc.bitcast(pj << 16, jnp.float32)
            yj = plsc.bitcast(pj & jnp.uint32(0xFFFF0000), jnp.float32)
            zj = plsc.load_gather(pz_v, [jv])
            sj = plsc.load_gather(sq_v, [jv])
            x2 = xj + xj
            y2 = yj + yj
            z2 = zj + zj

            @plsc.parallel_loop(
                0, N, L, unroll=8,
                carry=(bigv, jnp.zeros((L,), jnp.int32)))
            def sweep(i, carry):
                bv, bi = carry
                pxy = plsc.bitcast(px_v[pl.ds(i, L)], jnp.uint32)
                x = plsc.bitcast(pxy << 16, jnp.float32)
                y = plsc.bitcast(pxy & jnp.uint32(0xFFFF0000), jnp.float32)
                z = pz_v[pl.ds(i, L)]
                sv = fr_v[pl.ds(i, L)]
                m = fr_v[pl.ds(N + i, L)]
                d2 = (sv + sj) - (x * x2 + y * y2 + z * z2)
                m2 = jnp.minimum(m, d2)
                fr_v[pl.ds(N + i, L)] = m2
                better = m2 < bv
                bv = jnp.minimum(bv, m2)
                bi = jnp.where(better, jnp.full((L,), i, jnp.int32), bi)
                return bv, bi

            bv, bi = sweep
            m = jnp.min(bv)
            lane_v[...] = bi
            lane = plsc.all_reduce_ffs(bv == m)
            if lane.shape != (L,):
                lane = jnp.broadcast_to(lane, (L,))
            j = plsc.load_gather(lane_v, [lane]) + lane
            plsc.store_scatter(
                da_v, [jnp.full((L,), t, jnp.int32)],
                jnp.broadcast_to(m, (L,)), mask=lane0)
            return j

        lax.fori_loop(0, N - 1, step, jnp.zeros((L,), jnp.int32))
        plsc.store_scatter(
            da_v, [jnp.full((L,), N - 1, jnp.int32)], bigv,
            mask=lane0)

        def make_stage(src, dst, k, s):
            def body(i):
                g = jnp.full((L,), i, jnp.int32) + iota
                p = lax.bitwise_xor(g, jnp.int32(s))
                ag = src[pl.ds(i, L)]
                ap = plsc.load_gather(src, [p])
                up = lax.bitwise_and(g, jnp.int32(k)) == 0
                lowr = lax.bitwise_and(g, jnp.int32(s)) == 0
                tmin = up == lowr
                dst[pl.ds(i, L)] = jnp.where(
                    tmin, jnp.minimum(ag, ap), jnp.maximum(ag, ap))
            return body

        src, dst = da_v, db_v
        for kk in range(1, 11):
            k = 1 << kk
            for ss in range(kk - 1, -1, -1):
                plsc.parallel_loop(0, N, L, unroll=2)(
                    make_stage(src, dst, k, 1 << ss))
                src, dst = dst, src

        pltpu.sync_copy(src, out_hbm.at[row])


_sc_prim_sort = functools.partial(
    pl.kernel,
    mesh=plsc.VectorSubcoreMesh(core_axis_name="c", subcore_axis_name="s"),
    out_type=jax.ShapeDtypeStruct((NCLOUD * N,), jnp.float32),
    compiler_params=pltpu.CompilerParams(needs_layout_passes=False),
    scratch_types=[
        pltpu.VMEM((N,), jnp.float32),
        pltpu.VMEM((N,), jnp.float32),
        pltpu.VMEM((N,), jnp.float32),
        pltpu.VMEM((N,), jnp.float32),
        pltpu.VMEM((2 * N,), jnp.float32),
        pltpu.VMEM((N,), jnp.float32),
        pltpu.VMEM((N,), jnp.float32),
        pltpu.VMEM((L,), jnp.int32),
    ],
)(_sc_body)


def _finish_body(d2_ref, out_ref):
    d = jnp.sqrt(jnp.maximum(d2_ref[...], EPS))
    diff = d[0:8, :] - d[8:16, :]
    tl = jnp.sqrt(jnp.sum(diff * diff, axis=1))
    out_ref[...] = jnp.mean(tl).reshape(1, 1)


def kernel(gts, preds):
    pts = jnp.concatenate([gts, preds], 0)
    px = pts[:, :, 0].reshape(-1)
    py = pts[:, :, 1].reshape(-1)
    pz = pts[:, :, 2].reshape(-1)
    sorted_d2 = _sc_prim_sort(px, py, pz).reshape(NCLOUD, N)
    out = pl.pallas_call(
        _finish_body,
        out_shape=jax.ShapeDtypeStruct((1, 1), jnp.float32),
    )(sorted_d2)
    return out[0, 0]
